# baseline (speedup 1.0000x reference)
import jax
import jax.numpy as jnp
from jax.experimental import pallas as pl
from jax.experimental.pallas import tpu as pltpu


def _pos_kernel(row_ref, col_ref, out_ref, plane_ref, sems):
    h = row_ref.shape[0]
    w = col_ref.shape[0]
    f = row_ref.shape[1]
    b = out_ref.shape[0]
    top = jnp.broadcast_to(col_ref[...][None, :, :], (h, w, f))
    bot = jnp.broadcast_to(row_ref[...][:, None, :], (h, w, f))
    plane_ref[...] = jnp.concatenate([top, bot], axis=-1)
    copies = [
        pltpu.make_async_copy(plane_ref, out_ref.at[k], sems.at[k])
        for k in range(b)
    ]
    for c in copies:
        c.start()
    for c in copies:
        c.wait()


def kernel(x, row_embed, col_embed):
    b = x.shape[0]
    h, w = x.shape[-2], x.shape[-1]
    f = row_embed.shape[1]
    y = pl.pallas_call(
        _pos_kernel,
        in_specs=[
            pl.BlockSpec(memory_space=pltpu.VMEM),
            pl.BlockSpec(memory_space=pltpu.VMEM),
        ],
        out_specs=pl.BlockSpec(memory_space=pl.ANY),
        out_shape=jax.ShapeDtypeStruct((b, h, w, 2 * f), row_embed.dtype),
        scratch_shapes=[
            pltpu.VMEM((h, w, 2 * f), row_embed.dtype),
            pltpu.SemaphoreType.DMA((b,)),
        ],
    )(row_embed[:h], col_embed[:w])
    return jnp.transpose(y, (0, 3, 1, 2))


# speedup vs baseline: 1.3847x; 1.3847x over previous
import jax
import jax.numpy as jnp
from jax.experimental import pallas as pl


def _pos_kernel(row_ref, col_ref, out_ref):
    h = row_ref.shape[0]
    w = col_ref.shape[0]
    f = row_ref.shape[1]
    top = jnp.broadcast_to(col_ref[...][None, :, :], (h, w, f))
    bot = jnp.broadcast_to(row_ref[...][:, None, :], (h, w, f))
    plane = jnp.concatenate([top, bot], axis=-1)
    for k in range(out_ref.shape[0]):
        out_ref[k] = plane


def kernel(x, row_embed, col_embed):
    b = x.shape[0]
    h, w = x.shape[-2], x.shape[-1]
    f = row_embed.shape[1]
    y = pl.pallas_call(
        _pos_kernel,
        grid=(b // 2,),
        in_specs=[
            pl.BlockSpec((h, f), lambda i: (0, 0)),
            pl.BlockSpec((w, f), lambda i: (0, 0)),
        ],
        out_specs=pl.BlockSpec((2, h, w, 2 * f), lambda i: (i, 0, 0, 0)),
        out_shape=jax.ShapeDtypeStruct((b, h, w, 2 * f), row_embed.dtype),
    )(row_embed, col_embed)
    return jnp.transpose(y, (0, 3, 1, 2))
